# baseline (speedup 1.0000x reference)
import functools

import jax
import jax.numpy as jnp
from jax import lax
from jax.experimental import pallas as pl
from jax.experimental.pallas import tpu as pltpu
from jax.experimental.pallas import tpu_sc as plsc

EPS = 1e-5


def _gather_sum(table, idx_w, Np, C, n_acc, K1):
    info = plsc.get_sparse_core_info()
    NC, NS = info.num_cores, info.num_subcores
    NW = NC * NS
    per_w = Np // NW
    ncol = C // 16
    mesh = plsc.VectorSubcoreMesh(core_axis_name="c", subcore_axis_name="s")

    def body(table_ref, idx_ref, out_ref, idx_v, rows_v, out_v, sem0, sem1):
        w = lax.axis_index("s") * NC + lax.axis_index("c")
        pltpu.sync_copy(idx_ref.at[w], idx_v)
        sems = (sem0, sem1)

        for b in range(2):
            pltpu.async_copy(table_ref.at[idx_v.at[b]], rows_v.at[b], sems[b])

        def step(it, carry):
            i0 = it * 2
            for b in range(2):
                i = i0 + b
                pltpu.make_async_copy(table_ref.at[idx_v.at[i]],
                                      rows_v.at[b], sems[b]).wait()
                accs = [rows_v[b, 0, pl.ds(c * 16, 16)] for c in range(ncol)]
                for r in range(1, n_acc):
                    for c in range(ncol):
                        accs[c] = accs[c] + rows_v[b, r, pl.ds(c * 16, 16)]
                for c in range(ncol):
                    out_v[i, pl.ds(c * 16, 16)] = accs[c]
                nxt = jnp.minimum(i + 2, per_w - 1)
                pltpu.async_copy(table_ref.at[idx_v.at[nxt]],
                                 rows_v.at[b], sems[b])
            return carry

        lax.fori_loop(0, per_w // 2, step, 0)
        for b in range(2):
            pltpu.make_async_copy(table_ref.at[idx_v.at[0]],
                                  rows_v.at[b], sems[b]).wait()
        pltpu.sync_copy(out_v, out_ref.at[pl.ds(w * per_w, per_w)])

    f = pl.kernel(
        body,
        out_type=jax.ShapeDtypeStruct((Np, C), jnp.float32),
        mesh=mesh,
        scratch_types=[
            pltpu.VMEM((per_w, K1), jnp.int32),
            pltpu.VMEM((2, K1, C), jnp.float32),
            pltpu.VMEM((per_w, C), jnp.float32),
            pltpu.SemaphoreType.DMA,
            pltpu.SemaphoreType.DMA,
        ],
    )
    return f(table, idx_w)


def _tc1(s1, W1t, gm1, bt1, n_real):
    Np, C = s1.shape
    HID = W1t.shape[1]
    inv_n = 1.0 / float(n_real)

    def body(s_ref, w1_ref, g_ref, b_ref, o_ref):
        r = jnp.dot(s_ref[...], w1_ref[...], preferred_element_type=jnp.float32)
        mean = jnp.sum(r, axis=0, keepdims=True) * inv_n
        var = jnp.sum(r * r, axis=0, keepdims=True) * inv_n - mean * mean
        f1 = (r - mean) * lax.rsqrt(var + EPS) * g_ref[...] + b_ref[...]
        o_ref[...] = jnp.maximum(f1, 0.0)

    return pl.pallas_call(
        body, out_shape=jax.ShapeDtypeStruct((Np, HID), jnp.float32),
    )(s1, W1t, gm1, bt1)


def _tc2(s2, W2t, gm2, bt2, n_real):
    Np, C = s2.shape
    GF = W2t.shape[1]
    inv_n = 1.0 / float(n_real)

    def body(s_ref, w2_ref, g_ref, b_ref, o_ref):
        r = jnp.dot(s_ref[...], w2_ref[...], preferred_element_type=jnp.float32)
        mean = jnp.sum(r, axis=0, keepdims=True) * inv_n
        var = jnp.sum(r * r, axis=0, keepdims=True) * inv_n - mean * mean
        y = (r - mean) * lax.rsqrt(var + EPS) * g_ref[...] + b_ref[...]
        o_ref[...] = jnp.maximum(y, 0.0)

    return pl.pallas_call(
        body, out_shape=jax.ShapeDtypeStruct((Np, GF), jnp.float32),
    )(s2, W2t, gm2, bt2)


def kernel(fea, ring_n, pool_idx, W1, b1, g1, bt1, W2, b2, g2, bt2):
    M, C, F = fea.shape
    P, K = ring_n.shape[1], ring_n.shape[2]
    HID, GF = W1.shape[0], W2.shape[0]

    info = plsc.get_sparse_core_info()
    NW = info.num_cores * info.num_subcores
    n_real = M * P
    Np = -(-n_real // (8 * NW)) * (8 * NW)
    K1 = 40
    Z = M * F

    fea_t = fea.transpose(0, 2, 1).reshape(M * F, C)
    table1 = jnp.concatenate([fea_t, jnp.zeros((1, C), fea.dtype)], axis=0)

    mF = (jnp.arange(M, dtype=jnp.int32) * F)[:, None]
    selfr = jnp.arange(P, dtype=jnp.int32)[None, :] + mF
    ringr = ring_n + mF[:, :, None]
    idx_main = jnp.concatenate([selfr[..., None], ringr], axis=2)
    idx_main = idx_main.reshape(n_real, K + 1)
    idx_full = jnp.full((Np, K1), Z, jnp.int32).at[:n_real, :K + 1].set(idx_main)
    idx_w = idx_full.reshape(NW, Np // NW, K1)

    s1 = _gather_sum(table1, idx_w, Np, C, K + 1, K1)
    f1 = _tc1(s1, W1.T, g1.reshape(1, HID), bt1.reshape(1, HID),
              n_real)

    t2 = jnp.pad(f1[:n_real].reshape(M, P, HID),
                 ((0, 0), (0, F - P), (0, 0))).reshape(M * F, HID)
    table2 = jnp.concatenate([t2, jnp.zeros((1, HID), t2.dtype)], axis=0)
    s2 = _gather_sum(table2, idx_w, Np, HID, K + 1, K1)
    f2 = _tc2(s2, W2.T, g2.reshape(1, GF), bt2.reshape(1, GF), n_real)

    ph2 = jnp.pad(f2[:n_real].reshape(M, P, GF).transpose(0, 2, 1),
                  ((0, 0), (0, 0), (0, F - P)))
    return jnp.concatenate([fea, ph2], axis=1)


# speedup vs baseline: 1.2940x; 1.2940x over previous
import jax
import jax.numpy as jnp
from jax import lax
from jax.experimental import pallas as pl
from jax.experimental.pallas import tpu as pltpu
from jax.experimental.pallas import tpu_sc as plsc

EPS = 1e-5
NBUF = 8


def _gather_sum(table, idx_w, Np, C, n_acc, K1):
    info = plsc.get_sparse_core_info()
    NC, NS = info.num_cores, info.num_subcores
    NW = NC * NS
    per_w = Np // NW
    ncol = C // 16
    assert per_w % NBUF == 0
    mesh = plsc.VectorSubcoreMesh(core_axis_name="c", subcore_axis_name="s")

    def body(table_ref, idx_ref, out_ref, idx_v, rows_v, out_v, *sems):
        w = lax.axis_index("s") * NC + lax.axis_index("c")
        pltpu.sync_copy(idx_ref.at[w], idx_v)

        for b in range(NBUF):
            pltpu.async_copy(table_ref.at[idx_v.at[b]], rows_v.at[b], sems[b])

        def step(it, carry):
            i0 = it * NBUF
            for b in range(NBUF):
                i = i0 + b
                pltpu.make_async_copy(table_ref.at[idx_v.at[i]],
                                      rows_v.at[b], sems[b]).wait()
                accs = [rows_v[b, 0, pl.ds(c * 16, 16)] for c in range(ncol)]
                for r in range(1, n_acc):
                    for c in range(ncol):
                        accs[c] = accs[c] + rows_v[b, r, pl.ds(c * 16, 16)]
                for c in range(ncol):
                    out_v[i, pl.ds(c * 16, 16)] = accs[c]
                nxt = jnp.minimum(i + NBUF, per_w - 1)
                pltpu.async_copy(table_ref.at[idx_v.at[nxt]],
                                 rows_v.at[b], sems[b])
            return carry

        lax.fori_loop(0, per_w // NBUF, step, 0)
        for b in range(NBUF):
            pltpu.make_async_copy(table_ref.at[idx_v.at[0]],
                                  rows_v.at[b], sems[b]).wait()
        pltpu.sync_copy(out_v, out_ref.at[pl.ds(w * per_w, per_w)])

    f = pl.kernel(
        body,
        out_type=jax.ShapeDtypeStruct((Np, C), jnp.float32),
        mesh=mesh,
        scratch_types=[
            pltpu.VMEM((per_w, K1), jnp.int32),
            pltpu.VMEM((NBUF, K1, C), jnp.float32),
            pltpu.VMEM((per_w, C), jnp.float32),
        ] + [pltpu.SemaphoreType.DMA] * NBUF,
        compiler_params=pltpu.CompilerParams(use_tc_tiling_on_sc=False),
    )
    return f(table, idx_w)


def _tc1(s1, W1t, gm1, bt1, W2t, n_real):
    Np, C = s1.shape
    GF = W2t.shape[1]
    inv_n = 1.0 / float(n_real)

    def body(s_ref, w1_ref, g_ref, b_ref, w2_ref, o_ref):
        r = jnp.dot(s_ref[...], w1_ref[...], preferred_element_type=jnp.float32)
        mean = jnp.sum(r, axis=0, keepdims=True) * inv_n
        var = jnp.sum(r * r, axis=0, keepdims=True) * inv_n - mean * mean
        f1 = (r - mean) * lax.rsqrt(var + EPS) * g_ref[...] + b_ref[...]
        f1 = jnp.maximum(f1, 0.0)
        o_ref[...] = jnp.dot(f1, w2_ref[...], preferred_element_type=jnp.float32)

    return pl.pallas_call(
        body, out_shape=jax.ShapeDtypeStruct((Np, GF), jnp.float32),
    )(s1, W1t, gm1, bt1, W2t)


def _tc2(s2, gm2, bt2, n_real):
    Np, GF = s2.shape
    inv_n = 1.0 / float(n_real)

    def body(s_ref, g_ref, b_ref, o_ref):
        r = s_ref[...]
        mean = jnp.sum(r, axis=0, keepdims=True) * inv_n
        var = jnp.sum(r * r, axis=0, keepdims=True) * inv_n - mean * mean
        y = (r - mean) * lax.rsqrt(var + EPS) * g_ref[...] + b_ref[...]
        o_ref[...] = jnp.maximum(y, 0.0)

    return pl.pallas_call(
        body, out_shape=jax.ShapeDtypeStruct((Np, GF), jnp.float32),
    )(s2, gm2, bt2)


def kernel(fea, ring_n, pool_idx, W1, b1, g1, bt1, W2, b2, g2, bt2):
    M, C, F = fea.shape
    P, K = ring_n.shape[1], ring_n.shape[2]
    HID, GF = W1.shape[0], W2.shape[0]

    info = plsc.get_sparse_core_info()
    NW = info.num_cores * info.num_subcores
    n_real = M * P
    Np = -(-n_real // (NBUF * NW)) * (NBUF * NW)
    K1 = 40
    Z = M * F

    fea_t = fea.transpose(0, 2, 1).reshape(M * F, C)
    table1 = jnp.concatenate([fea_t, jnp.zeros((1, C), fea.dtype)], axis=0)

    mF = (jnp.arange(M, dtype=jnp.int32) * F)[:, None]
    selfr = jnp.arange(P, dtype=jnp.int32)[None, :] + mF
    ringr = ring_n + mF[:, :, None]
    idx_main = jnp.concatenate([selfr[..., None], ringr], axis=2)
    idx_main = idx_main.reshape(n_real, K + 1)
    idx_full = jnp.full((Np, K1), Z, jnp.int32).at[:n_real, :K + 1].set(idx_main)
    idx_w = idx_full.reshape(NW, Np // NW, K1)

    s1 = _gather_sum(table1, idx_w, Np, C, K + 1, K1)
    g2mat = _tc1(s1, W1.T, g1.reshape(1, HID), bt1.reshape(1, HID),
                 W2.T, n_real)

    t2 = jnp.pad(g2mat[:n_real].reshape(M, P, GF),
                 ((0, 0), (0, F - P), (0, 0))).reshape(M * F, GF)
    table2 = jnp.concatenate([t2, jnp.zeros((1, GF), t2.dtype)], axis=0)
    s2 = _gather_sum(table2, idx_w, Np, GF, K + 1, K1)
    f2 = _tc2(s2, g2.reshape(1, GF), bt2.reshape(1, GF), n_real)

    ph2 = jnp.pad(f2[:n_real].reshape(M, P, GF).transpose(0, 2, 1),
                  ((0, 0), (0, 0), (0, F - P)))
    return jnp.concatenate([fea, ph2], axis=1)


# speedup vs baseline: 18.9198x; 14.6212x over previous
import jax
import jax.numpy as jnp
from jax import lax
from jax.experimental import pallas as pl
from jax.experimental.pallas import tpu as pltpu
from jax.experimental.pallas import tpu_sc as plsc

EPS = 1e-5
NBUF = 4
CHUNK = 64


def _gather_sum(table, idx_w, Np, C, n_acc, K1, Ftab):
    info = plsc.get_sparse_core_info()
    NC, NS = info.num_cores, info.num_subcores
    per_w = Np // (NC * NS)
    ncol = C // 16
    rpt = Ftab // NS
    assert per_w % CHUNK == 0 and CHUNK % NBUF == 0
    assert Ftab % (NS * 8) == 0 and rpt % 8 == 0
    mesh = plsc.VectorSubcoreMesh(core_axis_name="c", subcore_axis_name="s")

    def body(tab_ref, idx_ref, out_ref, tab_s, idx_v, rows_v, out_v, *sems):
        c = lax.axis_index("c")
        s = lax.axis_index("s")
        pltpu.sync_copy(tab_ref.at[c, pl.ds(s * rpt, rpt)],
                        tab_s.at[pl.ds(s * rpt, rpt)])
        pltpu.sync_copy(idx_ref.at[c, s], idx_v)
        plsc.subcore_barrier()

        for b in range(NBUF):
            pltpu.async_copy(tab_s.at[idx_v.at[pl.ds(b * K1, K1)]],
                             rows_v.at[b], sems[b])

        def chunk_body(ch, carry0):
            def step(it, carry):
                i0 = ch * CHUNK + it * NBUF
                for b in range(NBUF):
                    i = i0 + b
                    pltpu.make_async_copy(tab_s.at[idx_v.at[pl.ds(0, K1)]],
                                          rows_v.at[b], sems[b]).wait()
                    accs = [rows_v[b, 0, pl.ds(k * 16, 16)]
                            for k in range(ncol)]
                    for r in range(1, n_acc):
                        for k in range(ncol):
                            accs[k] = accs[k] + rows_v[b, r, pl.ds(k * 16, 16)]
                    io = it * NBUF + b
                    for k in range(ncol):
                        out_v[io, pl.ds(k * 16, 16)] = accs[k]
                    nxt = jnp.minimum(i + NBUF, per_w - 1)
                    pltpu.async_copy(tab_s.at[idx_v.at[pl.ds(nxt * K1, K1)]],
                                     rows_v.at[b], sems[b])
                return carry

            lax.fori_loop(0, CHUNK // NBUF, step, 0)
            pltpu.sync_copy(out_v, out_ref.at[pl.ds(
                (c * NS + s) * per_w + ch * CHUNK, CHUNK)])
            return carry0

        lax.fori_loop(0, per_w // CHUNK, chunk_body, 0)
        for b in range(NBUF):
            pltpu.make_async_copy(tab_s.at[idx_v.at[pl.ds(0, K1)]],
                                  rows_v.at[b], sems[b]).wait()

    f = pl.kernel(
        body,
        out_type=jax.ShapeDtypeStruct((Np, C), jnp.float32),
        mesh=mesh,
        scratch_types=[
            pltpu.VMEM_SHARED((Ftab, C), jnp.float32),
            pltpu.VMEM((per_w * K1,), jnp.int32),
            pltpu.VMEM((NBUF, K1, C), jnp.float32),
            pltpu.VMEM((CHUNK, C), jnp.float32),
        ] + [pltpu.SemaphoreType.DMA] * NBUF,
    )
    return f(table, idx_w)


def _tct(x, blk):
    M, C, Fp = x.shape
    assert Fp % blk == 0

    def body(x_ref, o_ref):
        o_ref[0] = x_ref[0].T

    return pl.pallas_call(
        body,
        grid=(M, Fp // blk),
        in_specs=[pl.BlockSpec((1, C, blk), lambda m, j: (m, 0, j))],
        out_specs=pl.BlockSpec((1, blk, C), lambda m, j: (m, j, 0)),
        out_shape=jax.ShapeDtypeStruct((M, Fp, C), jnp.float32),
    )(x)


def _tc1(s1, W1t, gm1, bt1, n_real):
    Np, C = s1.shape
    HID = W1t.shape[1]
    inv_n = 1.0 / float(n_real)

    def body(s_ref, w1_ref, g_ref, b_ref, o_ref):
        r = jnp.dot(s_ref[...], w1_ref[...], preferred_element_type=jnp.float32)
        mean = jnp.sum(r, axis=0, keepdims=True) * inv_n
        var = jnp.sum(r * r, axis=0, keepdims=True) * inv_n - mean * mean
        f1 = (r - mean) * lax.rsqrt(var + EPS) * g_ref[...] + b_ref[...]
        o_ref[...] = jnp.maximum(f1, 0.0)

    return pl.pallas_call(
        body, out_shape=jax.ShapeDtypeStruct((Np, HID), jnp.float32),
    )(s1, W1t, gm1, bt1)


def _tc2(s2, W2t, gm2, bt2, n_real, M, Pp, P):
    Np, C = s2.shape
    GF = W2t.shape[1]
    inv_n = 1.0 / float(n_real)

    def body(s_ref, w2_ref, g_ref, b_ref, o_ref):
        r = jnp.dot(s_ref[...], w2_ref[...], preferred_element_type=jnp.float32)
        mean = jnp.sum(r, axis=0, keepdims=True) * inv_n
        var = jnp.sum(r * r, axis=0, keepdims=True) * inv_n - mean * mean
        y = (r - mean) * lax.rsqrt(var + EPS) * g_ref[...] + b_ref[...]
        y = jnp.maximum(y, 0.0)
        col = lax.broadcasted_iota(jnp.int32, (GF, Pp), 1)
        for m in range(M):
            ym = y[m * Pp:(m + 1) * Pp, :].T
            o_ref[m] = jnp.where(col < P, ym, 0.0)

    return pl.pallas_call(
        body, out_shape=jax.ShapeDtypeStruct((M, GF, Pp), jnp.float32),
    )(s2, W2t, gm2, bt2)


def kernel(fea, ring_n, pool_idx, W1, b1, g1, bt1, W2, b2, g2, bt2):
    M, C, F = fea.shape
    P, K = ring_n.shape[1], ring_n.shape[2]
    HID, GF = W1.shape[0], W2.shape[0]

    info = plsc.get_sparse_core_info()
    NC, NS = info.num_cores, info.num_subcores
    assert M == NC, "batch dim maps one mesh per SparseCore"
    per_w = -(-P // (NS * CHUNK)) * CHUNK
    Pp = NS * per_w
    Np = NC * Pp
    n_real = M * P
    K1 = 40
    Z = F
    Ftab = -(-(F + 1) // 512) * 512
    Ftab2 = -(-(P + 1) // (NS * 8)) * (NS * 8)

    fea_pad = jnp.pad(fea, ((0, 0), (0, 0), (0, Ftab - F)))
    table1 = _tct(fea_pad, 512)

    idx_full = jnp.full((M, Pp, K1), Z, jnp.int32)
    idx_main = jnp.concatenate(
        [jnp.broadcast_to(jnp.arange(P, dtype=jnp.int32)[None, :, None],
                          (M, P, 1)), ring_n], axis=2)
    idx_full = idx_full.at[:, :P, :K + 1].set(idx_main)
    idx_w = idx_full.reshape(M, NS, per_w * K1)

    s1 = _gather_sum(table1, idx_w, Np, C, K + 1, K1, Ftab)
    f1 = _tc1(s1, W1.T, g1.reshape(1, HID), bt1.reshape(1, HID),
              n_real)

    table2 = jnp.pad(f1.reshape(M, Pp, HID)[:, :P],
                     ((0, 0), (0, Ftab2 - P), (0, 0)))
    idx2_w = jnp.minimum(idx_w, P)
    s2 = _gather_sum(table2, idx2_w, Np, HID, K + 1, K1, Ftab2)
    f2t = _tc2(s2, W2.T, g2.reshape(1, GF), bt2.reshape(1, GF), n_real,
               M, Pp, P)

    ph2 = jnp.pad(f2t[:, :, :P], ((0, 0), (0, 0), (0, F - P)))
    return jnp.concatenate([fea, ph2], axis=1)


# speedup vs baseline: 19.4415x; 1.0276x over previous
import jax
import jax.numpy as jnp
from jax import lax
from jax.experimental import pallas as pl
from jax.experimental.pallas import tpu as pltpu
from jax.experimental.pallas import tpu_sc as plsc

EPS = 1e-5
NBUF = 4
CHUNK = 64


def _gather_sum(table, idx_w, Np, C, n_acc, K1, Ftab, KS):
    info = plsc.get_sparse_core_info()
    NC, NS = info.num_cores, info.num_subcores
    per_w = Np // (NC * NS)
    ncol = C // 16
    rpt = Ftab // NS
    assert per_w % CHUNK == 0 and CHUNK % NBUF == 0
    assert Ftab % (NS * 8) == 0 and rpt % 8 == 0
    mesh = plsc.VectorSubcoreMesh(core_axis_name="c", subcore_axis_name="s")

    def body(tab_ref, idx_ref, out_ref, tab_s, idx_v, rows_v, out_v, *sems):
        c = lax.axis_index("c")
        s = lax.axis_index("s")
        pltpu.sync_copy(tab_ref.at[c, pl.ds(s * rpt, rpt)],
                        tab_s.at[pl.ds(s * rpt, rpt)])
        pltpu.sync_copy(idx_ref.at[c, s], idx_v)
        plsc.subcore_barrier()

        for b in range(NBUF):
            pltpu.async_copy(tab_s.at[idx_v.at[pl.ds(b * K1, KS)]],
                             rows_v.at[b], sems[b])

        def chunk_body(ch, carry0):
            def step(it, carry):
                i0 = ch * CHUNK + it * NBUF
                for b in range(NBUF):
                    i = i0 + b
                    pltpu.make_async_copy(tab_s.at[idx_v.at[pl.ds(0, KS)]],
                                          rows_v.at[b], sems[b]).wait()
                    accs = [rows_v[b, 0, pl.ds(k * 16, 16)]
                            for k in range(ncol)]
                    for r in range(1, n_acc):
                        for k in range(ncol):
                            accs[k] = accs[k] + rows_v[b, r, pl.ds(k * 16, 16)]
                    io = it * NBUF + b
                    for k in range(ncol):
                        out_v[io, pl.ds(k * 16, 16)] = accs[k]
                    nxt = jnp.minimum(i + NBUF, per_w - 1)
                    pltpu.async_copy(tab_s.at[idx_v.at[pl.ds(nxt * K1, KS)]],
                                     rows_v.at[b], sems[b])
                return carry

            lax.fori_loop(0, CHUNK // NBUF, step, 0)
            pltpu.sync_copy(out_v, out_ref.at[pl.ds(
                (c * NS + s) * per_w + ch * CHUNK, CHUNK)])
            return carry0

        lax.fori_loop(0, per_w // CHUNK, chunk_body, 0)
        for b in range(NBUF):
            pltpu.make_async_copy(tab_s.at[idx_v.at[pl.ds(0, KS)]],
                                  rows_v.at[b], sems[b]).wait()

    f = pl.kernel(
        body,
        out_type=jax.ShapeDtypeStruct((Np, C), jnp.float32),
        mesh=mesh,
        scratch_types=[
            pltpu.VMEM_SHARED((Ftab, C), jnp.float32),
            pltpu.VMEM((per_w * K1,), jnp.int32),
            pltpu.VMEM((NBUF, KS, C), jnp.float32),
            pltpu.VMEM((CHUNK, C), jnp.float32),
        ] + [pltpu.SemaphoreType.DMA] * NBUF,
    )
    return f(table, idx_w)


def _tct(x, blk):
    M, C, Fp = x.shape
    assert Fp % blk == 0

    def body(x_ref, o_ref):
        o_ref[0] = x_ref[0].T

    return pl.pallas_call(
        body,
        grid=(M, Fp // blk),
        in_specs=[pl.BlockSpec((1, C, blk), lambda m, j: (m, 0, j))],
        out_specs=pl.BlockSpec((1, blk, C), lambda m, j: (m, j, 0)),
        out_shape=jax.ShapeDtypeStruct((M, Fp, C), jnp.float32),
    )(x)


def _tc1(s1, W1t, gm1, bt1, n_real):
    Np, C = s1.shape
    HID = W1t.shape[1]
    inv_n = 1.0 / float(n_real)

    def body(s_ref, w1_ref, g_ref, b_ref, o_ref):
        r = jnp.dot(s_ref[...], w1_ref[...], preferred_element_type=jnp.float32)
        mean = jnp.sum(r, axis=0, keepdims=True) * inv_n
        var = jnp.sum(r * r, axis=0, keepdims=True) * inv_n - mean * mean
        f1 = (r - mean) * lax.rsqrt(var + EPS) * g_ref[...] + b_ref[...]
        o_ref[...] = jnp.maximum(f1, 0.0)

    return pl.pallas_call(
        body, out_shape=jax.ShapeDtypeStruct((Np, HID), jnp.float32),
    )(s1, W1t, gm1, bt1)


def _tc2(s2, W2t, gm2, bt2, n_real, M, Pp, P):
    Np, C = s2.shape
    GF = W2t.shape[1]
    inv_n = 1.0 / float(n_real)

    def body(s_ref, w2_ref, g_ref, b_ref, o_ref):
        r = jnp.dot(s_ref[...], w2_ref[...], preferred_element_type=jnp.float32)
        mean = jnp.sum(r, axis=0, keepdims=True) * inv_n
        var = jnp.sum(r * r, axis=0, keepdims=True) * inv_n - mean * mean
        y = (r - mean) * lax.rsqrt(var + EPS) * g_ref[...] + b_ref[...]
        y = jnp.maximum(y, 0.0)
        col = lax.broadcasted_iota(jnp.int32, (GF, Pp), 1)
        for m in range(M):
            ym = y[m * Pp:(m + 1) * Pp, :].T
            o_ref[m] = jnp.where(col < P, ym, 0.0)

    return pl.pallas_call(
        body, out_shape=jax.ShapeDtypeStruct((M, GF, Pp), jnp.float32),
    )(s2, W2t, gm2, bt2)


def kernel(fea, ring_n, pool_idx, W1, b1, g1, bt1, W2, b2, g2, bt2):
    M, C, F = fea.shape
    P, K = ring_n.shape[1], ring_n.shape[2]
    HID, GF = W1.shape[0], W2.shape[0]

    info = plsc.get_sparse_core_info()
    NC, NS = info.num_cores, info.num_subcores
    assert M == NC, "batch dim maps one mesh per SparseCore"
    per_w = -(-P // (NS * CHUNK)) * CHUNK
    Pp = NS * per_w
    Np = NC * Pp
    n_real = M * P
    K1 = 40
    Z = F
    Ftab = -(-(F + 1) // 512) * 512
    Ftab2 = -(-(P + 1) // (NS * 8)) * (NS * 8)

    fea_pad = jnp.pad(fea, ((0, 0), (0, 0), (0, Ftab - F)))
    table1 = _tct(fea_pad, 512)

    idx_full = jnp.full((M, Pp, K1), Z, jnp.int32)
    idx_main = jnp.concatenate(
        [jnp.broadcast_to(jnp.arange(P, dtype=jnp.int32)[None, :, None],
                          (M, P, 1)), ring_n], axis=2)
    idx_full = idx_full.at[:, :P, :K + 1].set(idx_main)
    idx_w = idx_full.reshape(M, NS, per_w * K1)

    s1 = _gather_sum(table1, idx_w, Np, C, K + 1, K1, Ftab, K + 1)
    f1 = _tc1(s1, W1.T, g1.reshape(1, HID), bt1.reshape(1, HID),
              n_real)

    table2 = jnp.pad(f1.reshape(M, Pp, HID)[:, :P],
                     ((0, 0), (0, Ftab2 - P), (0, 0)))
    idx2_w = jnp.minimum(idx_w, P)
    s2 = _gather_sum(table2, idx2_w, Np, HID, K + 1, K1, Ftab2, K + 1)
    f2t = _tc2(s2, W2.T, g2.reshape(1, GF), bt2.reshape(1, GF), n_real,
               M, Pp, P)

    ph2 = jnp.pad(f2t[:, :, :P], ((0, 0), (0, 0), (0, F - P)))
    return jnp.concatenate([fea, ph2], axis=1)
